# Initial kernel scaffold; baseline (speedup 1.0000x reference)
import jax
import jax.numpy as jnp
from jax.experimental import pallas as pl


def kernel(x, edge_index, num_plants, pW1, pb1, pW2, pb2, qW1, qb1, qW2, qb2, cW0, cb0, cW1, cb1):
    raise NotImplementedError("write your pallas kernel here")


# speedup vs baseline: 17.4403x; 17.4403x over previous
import functools

import jax
import jax.numpy as jnp
from jax import lax
from jax.experimental import pallas as pl
from jax.experimental.pallas import tpu as pltpu
from jax.experimental.pallas import tpu_sc as plsc

N = 10000
E = 320000
D = 128
H = 128

NC = 2
NS = 16
NW = NC * NS
EPT = E // NW
CHUNK = 80
NCHUNK = EPT // CHUNK
RPT = 632
NPAD = NS * RPT
RPTH = 640
NH = NS * RPTH
WH = 16

_mesh = plsc.VectorSubcoreMesh(core_axis_name="c", subcore_axis_name="s")


@functools.partial(
    pl.kernel,
    out_type=jax.ShapeDtypeStruct((NC * NH,), jnp.float32),
    mesh=_mesh,
    scratch_types=[
        pltpu.VMEM((CHUNK,), jnp.float32),
        pltpu.VMEM((NCHUNK, CHUNK), jnp.int32),
        pltpu.VMEM_SHARED((NH,), jnp.float32),
    ],
)
def _sc_hist(dst_hbm, zeros_hbm, out_hbm, ones_v, didx, acc):
    c = lax.axis_index("c")
    s = lax.axis_index("s")
    wid = c * NS + s

    for i in range(CHUNK // 16):
        ones_v[pl.ds(i * 16, 16)] = jnp.ones((16,), jnp.float32)
    pltpu.sync_copy(dst_hbm.at[wid], didx)
    pltpu.sync_copy(zeros_hbm, acc.at[pl.ds(s * RPTH, RPTH)])
    plsc.subcore_barrier()

    def _step(t, _):
        pltpu.sync_copy(ones_v, acc.at[didx.at[t]], add=True)
        return 0

    lax.fori_loop(0, NCHUNK, _step, 0)
    plsc.subcore_barrier()
    pltpu.sync_copy(acc.at[pl.ds(s * RPTH, RPTH)],
                    out_hbm.at[pl.ds(c * NH + s * RPTH, RPTH)])


@functools.partial(
    pl.kernel,
    out_type=jax.ShapeDtypeStruct((NC, NPAD, H), jnp.float32),
    mesh=_mesh,
    scratch_types=[
        pltpu.VMEM((NCHUNK, CHUNK), jnp.int32),
        pltpu.VMEM((NCHUNK, CHUNK), jnp.int32),
        pltpu.VMEM((CHUNK, H), jnp.float32),
        pltpu.VMEM_SHARED((NPAD, H), jnp.float32),
        pltpu.SemaphoreType.DMA,
    ],
)
def _sc_agg(g_hbm, src_hbm, dst_hbm, zeros_hbm, out_hbm,
            sidx, didx, rows, acc, sem):
    c = lax.axis_index("c")
    s = lax.axis_index("s")
    wid = c * NS + s

    pltpu.sync_copy(src_hbm.at[wid], sidx)
    pltpu.sync_copy(dst_hbm.at[wid], didx)
    pltpu.sync_copy(zeros_hbm, acc.at[pl.ds(s * RPT, RPT)])
    plsc.subcore_barrier()

    def _step(t, _):
        pltpu.async_copy(g_hbm.at[sidx.at[t]], rows, sem).wait()
        pltpu.sync_copy(rows, acc.at[didx.at[t]], add=True)
        return 0

    lax.fori_loop(0, NCHUNK, _step, 0)
    plsc.subcore_barrier()
    pltpu.sync_copy(acc.at[pl.ds(s * RPT, RPT)],
                    out_hbm.at[c, pl.ds(s * RPT, RPT)])


BLK = RPT
_PREC = lax.Precision.HIGHEST


def _dot(a, b):
    return jnp.dot(a, b, preferred_element_type=jnp.float32, precision=_PREC)


def _encode_body(npl_ref, x_ref, dinv_ref, pW1_ref, pb1_ref, pW2_ref, pb2_ref,
                 qW1_ref, qb1_ref, qW2_ref, qb2_ref, cW0_ref, g1_ref):
    i = pl.program_id(0)
    rows = i * BLK + lax.broadcasted_iota(jnp.int32, (BLK, 1), 0)
    mask = rows < npl_ref[0, 0]
    x = x_ref[...]
    pe = _dot(jax.nn.relu(_dot(x, pW1_ref[...]) + pb1_ref[...]),
              pW2_ref[...]) + pb2_ref[...]
    qe = _dot(jax.nn.relu(_dot(x, qW1_ref[...]) + qb1_ref[...]),
              qW2_ref[...]) + qb2_ref[...]
    h0 = jnp.where(mask, pe, qe)
    g1_ref[...] = _dot(h0, cW0_ref[...]) * dinv_ref[:, 0:1]


def _combine_mm_body(agg_ref, g_ref, dinv_ref, b_ref, W_ref, out_ref):
    d0 = dinv_ref[:, 0:1]
    a = agg_ref[0] + agg_ref[1] + g_ref[...]
    h = jax.nn.relu(d0 * a + b_ref[...])
    out_ref[...] = _dot(h, W_ref[...]) * d0


def _final_body(agg_ref, g_ref, dinv_ref, b_ref, out_ref):
    d0 = dinv_ref[:, 0:1]
    a = agg_ref[0] + agg_ref[1] + g_ref[...]
    out_ref[...] = d0 * a + b_ref[...]


def _row_spec(w):
    return pl.BlockSpec((BLK, w), lambda i: (i, 0))


def _pair_spec(w):
    return pl.BlockSpec((NC, BLK, w), lambda i: (0, i, 0))


def _full_spec(shape):
    return pl.BlockSpec(shape, lambda i: (0,) * len(shape))


def _tc_encode(npl, x, dinv, pW1, pb1, pW2, pb2, qW1, qb1, qW2, qb2, cW0):
    w128 = _full_spec((D, H))
    b128 = _full_spec((1, H))
    return pl.pallas_call(
        _encode_body,
        grid=(NPAD // BLK,),
        in_specs=[
            pl.BlockSpec(memory_space=pltpu.SMEM),
            _row_spec(D), _row_spec(WH),
            w128, b128, w128, b128, w128, b128, w128, b128, w128,
        ],
        out_specs=_row_spec(H),
        out_shape=jax.ShapeDtypeStruct((NPAD, H), jnp.float32),
    )(npl, x, dinv, pW1, pb1, pW2, pb2, qW1, qb1, qW2, qb2, cW0)


def _tc_combine_mm(agg, g, dinv, b, W):
    return pl.pallas_call(
        _combine_mm_body,
        grid=(NPAD // BLK,),
        in_specs=[_pair_spec(H), _row_spec(H), _row_spec(WH),
                  _full_spec((1, H)), _full_spec((H, H))],
        out_specs=_row_spec(H),
        out_shape=jax.ShapeDtypeStruct((NPAD, H), jnp.float32),
    )(agg, g, dinv, b, W)


def _tc_final(agg, g, dinv, b):
    return pl.pallas_call(
        _final_body,
        grid=(NPAD // BLK,),
        in_specs=[_pair_spec(H), _row_spec(H), _row_spec(WH),
                  _full_spec((1, H))],
        out_specs=_row_spec(H),
        out_shape=jax.ShapeDtypeStruct((NPAD, H), jnp.float32),
    )(agg, g, dinv, b)


def kernel(x, edge_index, num_plants, pW1, pb1, pW2, pb2, qW1, qb1, qW2, qb2,
           cW0, cb0, cW1, cb1):
    src = edge_index[0].reshape(NW, NCHUNK, CHUNK)
    dst = edge_index[1].reshape(NW, NCHUNK, CHUNK)
    npl = jnp.asarray(num_plants, jnp.int32).reshape(1, 1)
    xp = jnp.pad(x, ((0, NPAD - N), (0, 0)))
    zeros = jnp.zeros((RPT, H), jnp.float32)
    zeros1d = jnp.zeros((RPTH,), jnp.float32)

    hist = _sc_hist(dst, zeros1d).reshape(NC, NH)[:, :NPAD]
    dinv = jax.lax.rsqrt(1.0 + hist[0] + hist[1])
    dinv16 = jnp.broadcast_to(dinv[:, None], (NPAD, WH))
    g1 = _tc_encode(npl, xp, dinv16,
                    pW1, pb1.reshape(1, H), pW2, pb2.reshape(1, H),
                    qW1, qb1.reshape(1, H), qW2, qb2.reshape(1, H), cW0)
    agg1 = _sc_agg(g1, src, dst, zeros)
    g2 = _tc_combine_mm(agg1, g1, dinv16, cb0.reshape(1, H), cW1)
    agg2 = _sc_agg(g2, src, dst, zeros)
    return _tc_final(agg2, g2, dinv16, cb1.reshape(1, H))[:N]
